# baseline (speedup 1.0000x reference)
import dataclasses

import jax
import jax.numpy as jnp
from jax.experimental import pallas as pl
from jax.experimental.pallas import tpu as pltpu
from jax.experimental.pallas import tpu_sc as plsc

NSEG = 256


def _sums_body(xb_ref, bb_ref, parts_ref, out_ref):
    i = pl.program_id(0)
    nsteps = pl.num_programs(0)

    @pl.when(i == 0)
    def _init():
        out_ref[...] = jnp.zeros_like(out_ref)

    b = bb_ref[0, 0, :]
    S = out_ref.shape[0]
    R = b.shape[0]
    seg = jax.lax.broadcasted_iota(jnp.int32, (S, R), 0)
    mbf = (b[None, :] == seg).astype(jnp.bfloat16)
    xh = xb_ref[...].astype(jnp.bfloat16)
    out_ref[...] += jnp.dot(mbf, xh, preferred_element_type=jnp.float32)

    @pl.when(i == nsteps - 1)
    def _fin():
        e = jnp.max(parts_ref[...], axis=0, keepdims=True)
        k = 1
        while k < S:
            shifted = jnp.concatenate(
                [jnp.zeros((1, k), jnp.float32), e[:, : S - k]], axis=1
            )
            e = jnp.maximum(e, shifted)
            k *= 2
        prev = jnp.concatenate(
            [jnp.zeros((1, 1), jnp.float32), e[:, : S - 1]], axis=1
        )
        rec = 1.0 / jnp.maximum(e - prev, 1.0)
        out_ref[...] *= jnp.transpose(rec)


def _mean_pool_tc(x, batch_i32, parts):
    N, F = x.shape
    R = 20000
    G = N // R
    batch3 = batch_i32.reshape(G, 1, R)
    n_tiles = parts.shape[0]
    return pl.pallas_call(
        _sums_body,
        grid=(G,),
        in_specs=[
            pl.BlockSpec((R, F), lambda i: (i, 0)),
            pl.BlockSpec((1, 1, R), lambda i: (i, 0, 0)),
            pl.BlockSpec((n_tiles, NSEG), lambda i: (0, 0)),
        ],
        out_specs=pl.BlockSpec((NSEG, F), lambda i: (0, 0)),
        out_shape=jax.ShapeDtypeStruct((NSEG, F), jnp.float32),
    )(x, batch3, parts)


def _segment_ends(batch_pad, n_tiles, per_tile):
    mesh = plsc.VectorSubcoreMesh(core_axis_name="c", subcore_axis_name="s")
    buf_len = per_tile + 16
    ends_len = NSEG + 16

    cp = pltpu.CompilerParams()
    if "needs_layout_passes" in pltpu.CompilerParams.__dataclass_fields__:
        cp = dataclasses.replace(cp, needs_layout_passes=False)

    @pl.kernel(
        out_type=jax.ShapeDtypeStruct((n_tiles, NSEG), jnp.float32),
        mesh=mesh,
        compiler_params=cp,
        scratch_types=[
            pltpu.VMEM((buf_len,), jnp.int32),
            pltpu.VMEM((ends_len,), jnp.float32),
            pltpu.SemaphoreType.DMA,
        ],
    )
    def ends_kernel(b_hbm, out_hbm, buf, ends, sem):
        c = jax.lax.axis_index("c")
        s = jax.lax.axis_index("s")
        tile = c * 16 + s
        base = tile * per_tile

        @pl.loop(0, ends_len, step=16)
        def _zero(i):
            ends[pl.ds(i, 16)] = jnp.zeros((16,), jnp.float32)

        pltpu.async_copy(b_hbm.at[pl.ds(base, buf_len)], buf, sem).wait()

        lane = jax.lax.iota(jnp.int32, 16)

        @pl.loop(0, per_tile, step=16)
        def _scan(i):
            b0 = buf[pl.ds(i, 16)]
            b1 = buf[pl.ds(i + 1, 16)]
            posf = (lane + (base + i + 1)).astype(jnp.float32)
            plsc.store_scatter(ends, [b0], posf, mask=b0 != b1)

        pltpu.async_copy(ends.at[pl.ds(0, NSEG)], out_hbm.at[tile], sem).wait()

    return ends_kernel(batch_pad)


def kernel(x, batch):
    N, F = x.shape
    batch_i32 = batch.astype(jnp.int32)
    n_tiles = 32
    per_tile = -(-N // (n_tiles * 16)) * 16
    padded = n_tiles * per_tile + 16
    batch_pad = jnp.concatenate(
        [batch_i32, jnp.full((padded - N,), NSEG, dtype=jnp.int32)]
    )
    parts = _segment_ends(batch_pad, n_tiles, per_tile)
    return _mean_pool_tc(x, batch_i32, parts)


# speedup vs baseline: 1.0490x; 1.0490x over previous
import dataclasses

import jax
import jax.numpy as jnp
from jax.experimental import pallas as pl
from jax.experimental.pallas import tpu as pltpu
from jax.experimental.pallas import tpu_sc as plsc

NSEG = 256


def _sums_body(xb_ref, bb_ref, parts_ref, out_ref):
    i = pl.program_id(0)
    nsteps = pl.num_programs(0)

    @pl.when(i == 0)
    def _init():
        out_ref[...] = jnp.zeros_like(out_ref)

    b = bb_ref[0, 0, :]
    S = out_ref.shape[0]
    R = b.shape[0]
    seg = jax.lax.broadcasted_iota(jnp.int32, (S, R), 0)
    mbf = (b[None, :] == seg).astype(jnp.bfloat16)
    xh = xb_ref[...].astype(jnp.bfloat16)
    out_ref[...] += jnp.dot(mbf, xh, preferred_element_type=jnp.float32)

    @pl.when(i == nsteps - 1)
    def _fin():
        e = jnp.max(parts_ref[...], axis=0, keepdims=True)
        k = 1
        while k < S:
            shifted = jnp.concatenate(
                [jnp.zeros((1, k), jnp.float32), e[:, : S - k]], axis=1
            )
            e = jnp.maximum(e, shifted)
            k *= 2
        prev = jnp.concatenate(
            [jnp.zeros((1, 1), jnp.float32), e[:, : S - 1]], axis=1
        )
        rec = 1.0 / jnp.maximum(e - prev, 1.0)
        out_ref[...] *= jnp.transpose(rec)


def _mean_pool_tc(x, batch_i32, parts):
    N, F = x.shape
    R = 10000
    G = N // R
    batch3 = batch_i32.reshape(G, 1, R)
    n_tiles = parts.shape[0]
    return pl.pallas_call(
        _sums_body,
        grid=(G,),
        in_specs=[
            pl.BlockSpec((R, F), lambda i: (i, 0)),
            pl.BlockSpec((1, 1, R), lambda i: (i, 0, 0)),
            pl.BlockSpec((n_tiles, NSEG), lambda i: (0, 0)),
        ],
        out_specs=pl.BlockSpec((NSEG, F), lambda i: (0, 0)),
        out_shape=jax.ShapeDtypeStruct((NSEG, F), jnp.float32),
    )(x, batch3, parts)


def _segment_ends(batch_pad, n_tiles, per_tile):
    mesh = plsc.VectorSubcoreMesh(core_axis_name="c", subcore_axis_name="s")
    buf_len = per_tile + 16
    ends_len = NSEG + 16

    cp = pltpu.CompilerParams()
    if "needs_layout_passes" in pltpu.CompilerParams.__dataclass_fields__:
        cp = dataclasses.replace(cp, needs_layout_passes=False)

    @pl.kernel(
        out_type=jax.ShapeDtypeStruct((n_tiles, NSEG), jnp.float32),
        mesh=mesh,
        compiler_params=cp,
        scratch_types=[
            pltpu.VMEM((buf_len,), jnp.int32),
            pltpu.VMEM((ends_len,), jnp.float32),
            pltpu.SemaphoreType.DMA,
        ],
    )
    def ends_kernel(b_hbm, out_hbm, buf, ends, sem):
        c = jax.lax.axis_index("c")
        s = jax.lax.axis_index("s")
        tile = c * 16 + s
        base = tile * per_tile

        @pl.loop(0, ends_len, step=16)
        def _zero(i):
            ends[pl.ds(i, 16)] = jnp.zeros((16,), jnp.float32)

        pltpu.async_copy(b_hbm.at[pl.ds(base, buf_len)], buf, sem).wait()

        lane = jax.lax.iota(jnp.int32, 16)

        @pl.loop(0, per_tile, step=16)
        def _scan(i):
            b0 = buf[pl.ds(i, 16)]
            b1 = buf[pl.ds(i + 1, 16)]
            posf = (lane + (base + i + 1)).astype(jnp.float32)
            plsc.store_scatter(ends, [b0], posf, mask=b0 != b1)

        pltpu.async_copy(ends.at[pl.ds(0, NSEG)], out_hbm.at[tile], sem).wait()

    return ends_kernel(batch_pad)


def kernel(x, batch):
    N, F = x.shape
    batch_i32 = batch.astype(jnp.int32)
    n_tiles = 32
    per_tile = -(-N // (n_tiles * 16)) * 16
    padded = n_tiles * per_tile + 16
    batch_pad = jnp.concatenate(
        [batch_i32, jnp.full((padded - N,), NSEG, dtype=jnp.int32)]
    )
    parts = _segment_ends(batch_pad, n_tiles, per_tile)
    return _mean_pool_tc(x, batch_i32, parts)


# speedup vs baseline: 1.6238x; 1.5480x over previous
import jax
import jax.numpy as jnp
from jax.experimental import pallas as pl
from jax.experimental.pallas import tpu as pltpu

NSEG = 256


def _mean_pool_body(xb_ref, bb_ref, out_ref, cnt_ref):
    i = pl.program_id(0)
    nsteps = pl.num_programs(0)

    @pl.when(i == 0)
    def _init():
        out_ref[...] = jnp.zeros_like(out_ref)
        cnt_ref[...] = jnp.zeros_like(cnt_ref)

    b = bb_ref[0, 0, :]
    S = out_ref.shape[0]
    R = b.shape[0]
    seg = jax.lax.broadcasted_iota(jnp.int32, (S, R), 0)
    mask = b[None, :] == seg
    mbf = mask.astype(jnp.bfloat16)
    xh = xb_ref[...].astype(jnp.bfloat16)
    out_ref[...] += jnp.dot(mbf, xh, preferred_element_type=jnp.float32)
    cnt_ref[...] += jnp.sum(mask.astype(jnp.float32), axis=1, keepdims=True)

    @pl.when(i == nsteps - 1)
    def _fin():
        out_ref[...] = out_ref[...] / jnp.maximum(cnt_ref[...], 1.0)


def kernel(x, batch):
    N, F = x.shape
    R = 10000
    G = N // R
    batch3 = batch.astype(jnp.int32).reshape(G, 1, R)

    return pl.pallas_call(
        _mean_pool_body,
        grid=(G,),
        in_specs=[
            pl.BlockSpec((R, F), lambda i: (i, 0)),
            pl.BlockSpec((1, 1, R), lambda i: (i, 0, 0)),
        ],
        out_specs=pl.BlockSpec((NSEG, F), lambda i: (0, 0)),
        out_shape=jax.ShapeDtypeStruct((NSEG, F), jnp.float32),
        scratch_shapes=[pltpu.VMEM((NSEG, 1), jnp.float32)],
    )(x, batch3)
